# baseline (speedup 1.0000x reference)
import jax
import jax.numpy as jnp
from jax import lax
from jax.experimental import pallas as pl
from jax.experimental.pallas import tpu as pltpu

B, S, D, K = 4, 2048, 768, 1024
N = B * S
BM = 512
BIG = 3.0e38


def _tc_body(x_ref, a_ref, abf_ref, basin_ref, w_ref, b_ref, c_ref,
             out_ref, kprm_ref):
    @pl.when(pl.program_id(0) == 0)
    def _init():
        basin = jax.nn.softplus(basin_ref[...]) + 0.1
        am = a_ref[...]
        kprm_ref[0:1, :] = 1.0 / (basin * basin)
        kprm_ref[1:2, :] = jnp.sum(am * am, axis=1)[None, :]

    xb = x_ref[...]
    xp = lax.dot_general(xb, w_ref[...], (((1,), (1,)), ((), ())),
                         preferred_element_type=jnp.float32)
    xp = xp + b_ref[...]
    sc = lax.dot_general(xp, a_ref[...], (((1,), (1,)), ((), ())),
                         preferred_element_type=jnp.float32)
    x2 = jnp.sum(xp * xp, axis=1, keepdims=True)
    ib2 = kprm_ref[0:1, :]
    a2 = kprm_ref[1:2, :]
    nk = jnp.maximum(x2 + a2 - 2.0 * sc, 1e-12) * ib2

    m0 = jnp.min(nk, axis=1, keepdims=True)
    cm0 = nk == m0
    r1 = jnp.where(cm0, BIG, nk)
    m1 = jnp.min(r1, axis=1, keepdims=True)
    cm1 = r1 == m1
    r2 = jnp.where(cm1, BIG, r1)
    m2 = jnp.min(r2, axis=1, keepdims=True)
    cm2 = r2 == m2

    m012 = jnp.concatenate([m0, m1, m2], axis=1)
    asel = jnp.exp(jnp.clip(-jnp.sqrt(m012), -50.0, 50.0))
    esel = jnp.exp(asel - asel[:, 0:1])
    itot = 1.0 / jnp.sum(esel, axis=1, keepdims=True)
    w0 = esel[:, 0:1] * itot
    w1 = esel[:, 1:2] * itot
    w2 = esel[:, 2:3] * itot

    zero = jnp.float32(0.0)
    oh = jnp.where(cm0, w0,
                   jnp.where(cm1, w1,
                             jnp.where(cm2, w2, zero))).astype(jnp.bfloat16)
    mix = lax.dot_general(oh, abf_ref[...], (((1,), (0,)), ((), ())),
                          preferred_element_type=jnp.float32)

    c1 = c_ref[0, 0]
    out_ref[...] = c1 * xb + mix


@jax.jit
def kernel(x, attractors, basin_strengths, W, b):
    strength = jax.nn.sigmoid(jnp.float32(0.1))
    coef = jnp.stack([1.0 - strength, strength]).reshape(1, 2)
    x2d = x.reshape(N, D)
    out = pl.pallas_call(
        _tc_body,
        grid=(N // BM,),
        in_specs=[
            pl.BlockSpec((BM, D), lambda i: (i, 0)),
            pl.BlockSpec((K, D), lambda i: (0, 0)),
            pl.BlockSpec((K, D), lambda i: (0, 0)),
            pl.BlockSpec((1, K), lambda i: (0, 0)),
            pl.BlockSpec((D, D), lambda i: (0, 0)),
            pl.BlockSpec((1, D), lambda i: (0, 0)),
            pl.BlockSpec((1, 2), lambda i: (0, 0)),
        ],
        out_specs=pl.BlockSpec((BM, D), lambda i: (i, 0)),
        out_shape=jax.ShapeDtypeStruct((N, D), jnp.float32),
        scratch_shapes=[pltpu.VMEM((2, K), jnp.float32)],
    )(x2d, attractors, (strength * attractors).astype(jnp.bfloat16),
      basin_strengths.reshape(1, K), W, b.reshape(1, D), coef)
    return out.reshape(B, S, D)


# speedup vs baseline: 1.2264x; 1.2264x over previous
import jax
import jax.numpy as jnp
from jax import lax
from jax.experimental import pallas as pl
from jax.experimental.pallas import tpu as pltpu

B, S, D, K = 4, 2048, 768, 1024
N = B * S
BM = 1024
BIG = 3.0e38


def _tc_body(x_ref, a_ref, abf_ref, basin_ref, w_ref, b_ref, c_ref,
             out_ref, kprm_ref):
    @pl.when(pl.program_id(0) == 0)
    def _init():
        basin = jax.nn.softplus(basin_ref[...]) + 0.1
        am = a_ref[...]
        kprm_ref[0:1, :] = 1.0 / (basin * basin)
        kprm_ref[1:2, :] = jnp.sum(am * am, axis=1)[None, :]

    xb = x_ref[...]
    xp = lax.dot_general(xb, w_ref[...], (((1,), (1,)), ((), ())),
                         preferred_element_type=jnp.float32)
    xp = xp + b_ref[...]
    sc = lax.dot_general(xp, a_ref[...], (((1,), (1,)), ((), ())),
                         preferred_element_type=jnp.float32)
    x2 = jnp.sum(xp * xp, axis=1, keepdims=True)
    ib2 = kprm_ref[0:1, :]
    a2 = kprm_ref[1:2, :]
    nk = jnp.maximum(x2 + a2 - 2.0 * sc, 1e-12) * ib2

    m0 = jnp.min(nk, axis=1, keepdims=True)
    cm0 = nk == m0
    r1 = jnp.where(cm0, BIG, nk)
    m1 = jnp.min(r1, axis=1, keepdims=True)
    cm1 = r1 == m1
    r2 = jnp.where(cm1, BIG, r1)
    m2 = jnp.min(r2, axis=1, keepdims=True)
    cm2 = r2 == m2

    a0 = jnp.exp(jnp.clip(-jnp.sqrt(m0), -50.0, 50.0))
    a1 = jnp.exp(jnp.clip(-jnp.sqrt(m1), -50.0, 50.0))
    a2s = jnp.exp(jnp.clip(-jnp.sqrt(m2), -50.0, 50.0))
    e1 = jnp.exp(a1 - a0)
    e2 = jnp.exp(a2s - a0)
    itot = 1.0 / (1.0 + e1 + e2)
    w0 = itot
    w1 = e1 * itot
    w2 = e2 * itot

    zero = jnp.float32(0.0)
    oh = jnp.where(cm0, w0,
                   jnp.where(cm1, w1,
                             jnp.where(cm2, w2, zero))).astype(jnp.bfloat16)
    mix = lax.dot_general(oh, abf_ref[...], (((1,), (0,)), ((), ())),
                          preferred_element_type=jnp.float32)

    c1 = c_ref[0, 0]
    out_ref[...] = c1 * xb + mix


@jax.jit
def kernel(x, attractors, basin_strengths, W, b):
    strength = jax.nn.sigmoid(jnp.float32(0.1))
    coef = jnp.stack([1.0 - strength, strength]).reshape(1, 2)
    x2d = x.reshape(N, D)
    out = pl.pallas_call(
        _tc_body,
        grid=(N // BM,),
        in_specs=[
            pl.BlockSpec((BM, D), lambda i: (i, 0)),
            pl.BlockSpec((K, D), lambda i: (0, 0)),
            pl.BlockSpec((K, D), lambda i: (0, 0)),
            pl.BlockSpec((1, K), lambda i: (0, 0)),
            pl.BlockSpec((D, D), lambda i: (0, 0)),
            pl.BlockSpec((1, D), lambda i: (0, 0)),
            pl.BlockSpec((1, 2), lambda i: (0, 0)),
        ],
        out_specs=pl.BlockSpec((BM, D), lambda i: (i, 0)),
        out_shape=jax.ShapeDtypeStruct((N, D), jnp.float32),
        scratch_shapes=[pltpu.VMEM((2, K), jnp.float32)],
    )(x2d, attractors, (strength * attractors).astype(jnp.bfloat16),
      basin_strengths.reshape(1, K), W, b.reshape(1, D), coef)
    return out.reshape(B, S, D)
